# baseline (speedup 1.0000x reference)
import functools

import jax
import jax.numpy as jnp
from jax import lax
from jax.experimental import pallas as pl
from jax.experimental.pallas import tpu as pltpu
from jax.experimental.pallas import tpu_sc as plsc

ROWS, COLS = 16384, 200
VOCAB, DIM = 64, 64
NPAIR = ROWS * COLS // 2
PDIM = 2 * DIM
NW = 32
NSUB = 16
P_PER_W = NPAIR // NW
CHUNK = 320
N_CHUNKS = P_PER_W // CHUNK
N_PAIRS = N_CHUNKS // 2
PT_ROWS = VOCAB * VOCAB
PT_PER_SUB = PT_ROWS // NSUB


def _make_kernel():
    mesh = plsc.VectorSubcoreMesh(core_axis_name="c", subcore_axis_name="s")

    @functools.partial(
        pl.kernel,
        mesh=mesh,
        out_type=jax.ShapeDtypeStruct((NPAIR, PDIM), jnp.float32),
        scratch_types=[
            pltpu.VMEM((CHUNK,), jnp.int32),
            pltpu.VMEM((CHUNK,), jnp.int32),
            pltpu.VMEM((CHUNK, PDIM), jnp.float32),
            pltpu.VMEM((CHUNK, PDIM), jnp.float32),
            pltpu.VMEM_SHARED((PT_ROWS, PDIM), jnp.float32),
            pltpu.SemaphoreType.DMA,
            pltpu.SemaphoreType.DMA,
            pltpu.SemaphoreType.DMA,
            pltpu.SemaphoreType.DMA,
            pltpu.SemaphoreType.DMA,
            pltpu.SemaphoreType.DMA,
        ],
        compiler_params=pltpu.CompilerParams(use_tc_tiling_on_sc=False),
    )
    def gather_kernel(pidx_hbm, ptab_hbm, out_hbm,
                      idx0, idx1, rows0, rows1, ptab_v,
                      si0, si1, sg0, sg1, so0, so1):
        sid = lax.axis_index("s")
        wid = sid * 2 + lax.axis_index("c")
        w_base = wid * P_PER_W
        idx_v = (idx0, idx1)
        rows_v = (rows0, rows1)
        sem_i = (si0, si1)
        sem_g = (sg0, sg1)
        sem_o = (so0, so1)

        pt0 = sid * PT_PER_SUB
        pltpu.sync_copy(ptab_hbm.at[pl.ds(pt0, PT_PER_SUB)],
                        ptab_v.at[pl.ds(pt0, PT_PER_SUB)])
        plsc.subcore_barrier()

        for b in range(2):
            pltpu.async_copy(
                pidx_hbm.at[pl.ds(w_base + b * CHUNK, CHUNK)], idx_v[b],
                sem_i[b])

        def body(j, _):
            for b in range(2):
                base = w_base + (2 * j + b) * CHUNK
                pltpu.make_async_copy(
                    pidx_hbm.at[pl.ds(w_base, CHUNK)], idx_v[b],
                    sem_i[b]).wait()

                @pl.when(j >= 1)
                def _():
                    pltpu.make_async_copy(
                        rows_v[b], out_hbm.at[pl.ds(w_base, CHUNK)],
                        sem_o[b]).wait()

                pltpu.async_copy(
                    ptab_v.at[idx_v[b]], rows_v[b], sem_g[b]).wait()

                @pl.when(j < N_PAIRS - 1)
                def _():
                    pltpu.async_copy(
                        pidx_hbm.at[pl.ds(base + 2 * CHUNK, CHUNK)],
                        idx_v[b], sem_i[b])

                pltpu.async_copy(
                    rows_v[b], out_hbm.at[pl.ds(base, CHUNK)], sem_o[b])
            return 0

        lax.fori_loop(0, N_PAIRS, body, 0)

        for b in range(2):
            pltpu.make_async_copy(
                rows_v[b], out_hbm.at[pl.ds(w_base, CHUNK)],
                sem_o[b]).wait()

    return gather_kernel


_gather = _make_kernel()


@jax.jit
def kernel(X, table):
    pidx = (X[:, 0::2] * VOCAB + X[:, 1::2]).reshape(NPAIR)
    ptab = jnp.concatenate(
        [jnp.repeat(table, VOCAB, axis=0), jnp.tile(table, (VOCAB, 1))],
        axis=1)
    flat = _gather(pidx, ptab)
    return flat.reshape(ROWS, COLS, DIM)


# speedup vs baseline: 1.0251x; 1.0251x over previous
import functools

import jax
import jax.numpy as jnp
from jax import lax
from jax.experimental import pallas as pl
from jax.experimental.pallas import tpu as pltpu
from jax.experimental.pallas import tpu_sc as plsc

ROWS, COLS = 16384, 200
VOCAB, DIM = 64, 64
NW = 32
R_PER_W = ROWS // NW
G = 4
N_CHUNKS = R_PER_W // G
N_PAIRS = N_CHUNKS // 2


def _make_kernel():
    mesh = plsc.VectorSubcoreMesh(core_axis_name="c", subcore_axis_name="s")

    @functools.partial(
        pl.kernel,
        mesh=mesh,
        out_type=jax.ShapeDtypeStruct((ROWS, COLS, DIM), jnp.float32),
        scratch_types=[
            pltpu.VMEM((G, COLS), jnp.int32),
            pltpu.VMEM((G, COLS), jnp.int32),
            pltpu.VMEM((G, COLS, DIM), jnp.float32),
            pltpu.VMEM((G, COLS, DIM), jnp.float32),
            pltpu.VMEM_SHARED((VOCAB, DIM), jnp.float32),
            pltpu.SemaphoreType.DMA,
            pltpu.SemaphoreType.DMA,
            pltpu.SemaphoreType.DMA,
            pltpu.SemaphoreType.DMA,
            pltpu.SemaphoreType.DMA,
            pltpu.SemaphoreType.DMA,
        ],
        compiler_params=pltpu.CompilerParams(use_tc_tiling_on_sc=False),
    )
    def gather_kernel(idx_hbm, table_hbm, out_hbm,
                      idx0, idx1, rows0, rows1, table_v,
                      si0, si1, sg0, sg1, so0, so1):
        wid = lax.axis_index("s") * 2 + lax.axis_index("c")
        w_row = wid * R_PER_W
        idx_v = (idx0, idx1)
        rows_v = (rows0, rows1)
        sem_i = (si0, si1)
        sem_g = (sg0, sg1)
        sem_o = (so0, so1)

        pltpu.sync_copy(table_hbm, table_v)

        for b in range(2):
            pltpu.async_copy(
                idx_hbm.at[pl.ds(w_row + b * G, G), :], idx_v[b], sem_i[b])

        def body(j, _):
            for b in range(2):
                row0 = w_row + (2 * j + b) * G
                pltpu.make_async_copy(
                    idx_hbm.at[pl.ds(w_row, G), :], idx_v[b],
                    sem_i[b]).wait()

                @pl.when(j >= 1)
                def _():
                    pltpu.make_async_copy(
                        rows_v[b], out_hbm.at[pl.ds(w_row, G)],
                        sem_o[b]).wait()

                for g in range(G):
                    pltpu.async_copy(
                        table_v.at[idx_v[b].at[g]],
                        rows_v[b].at[g],
                        sem_g[b])
                for g in range(G):
                    pltpu.make_async_copy(
                        table_v.at[idx_v[b].at[g]],
                        rows_v[b].at[0],
                        sem_g[b]).wait()

                @pl.when(j < N_PAIRS - 1)
                def _():
                    pltpu.async_copy(
                        idx_hbm.at[pl.ds(row0 + 2 * G, G), :],
                        idx_v[b], sem_i[b])

                pltpu.async_copy(
                    rows_v[b], out_hbm.at[pl.ds(row0, G)], sem_o[b])
            return 0

        lax.fori_loop(0, N_PAIRS, body, 0)

        for b in range(2):
            pltpu.make_async_copy(
                rows_v[b], out_hbm.at[pl.ds(w_row, G)], sem_o[b]).wait()

    return gather_kernel


_gather = _make_kernel()


@jax.jit
def kernel(X, table):
    return _gather(X, table)
